# baseline (speedup 1.0000x reference)
import jax
import jax.numpy as jnp
from jax import lax
from jax.experimental import pallas as pl
from jax.experimental.pallas import tpu as pltpu
from jax.experimental.pallas import tpu_sc as plsc

L = 16
NC = 2
NS = 16
NW = NC * NS
HID = 128
SEQ = 200
K = 128
EPS = 1e-12

def _rsqrt_scalar(x):
    i = lax.bitcast_convert_type(x, jnp.int32)
    i = jnp.int32(0x5F3759DF) - lax.shift_right_logical(i, 1)
    y = lax.bitcast_convert_type(i, jnp.float32)
    y = y * (1.5 - 0.5 * x * y * y)
    y = y * (1.5 - 0.5 * x * y * y)
    return y


def _body(ids_hbm, tt_hbm, word_hbm, comb_hbm, gamma_hbm, beta_hbm, out_hbm,
          idx_v, ci_v, tt_v, wrows, obuf, comb_v, gb_v,
          sem_i, sem_w, sem_o):
    wid = lax.axis_index("s") * NC + lax.axis_index("c")
    tokens_per_w = ids_hbm.shape[0] // NW
    nchunks = tokens_per_w // K
    base = wid * tokens_per_w

    pltpu.sync_copy(gamma_hbm, gb_v.at[0])
    pltpu.sync_copy(beta_hbm, gb_v.at[1])
    pltpu.sync_copy(comb_hbm, comb_v)
    gregs = [gb_v[0, pl.ds(L * j, L)] for j in range(HID // L)]
    bregs = [gb_v[1, pl.ds(L * j, L)] for j in range(HID // L)]
    iota = lax.iota(jnp.int32, L)

    acc = None
    for j in range(HID // L):
        ok = jnp.logical_and(gregs[j] == 1.0, bregs[j] == 0.0)
        acc = ok if acc is None else jnp.logical_and(acc, ok)
    gb_trivial = jnp.all(acc)

    def ids_copies(c, b):
        g = base + c * K
        return (pltpu.make_async_copy(ids_hbm.at[pl.ds(g, K)], idx_v.at[b],
                                      sem_i[b]),
                pltpu.make_async_copy(tt_hbm.at[pl.ds(g, K)], tt_v.at[b],
                                      sem_i[b]))

    def gather_copy(b):
        return pltpu.make_async_copy(word_hbm.at[idx_v.at[b]], wrows.at[b],
                                     sem_w[b])

    def out_copy(c, b):
        g = base + c * K
        return pltpu.make_async_copy(obuf.at[b], out_hbm.at[pl.ds(g, K)],
                                     sem_o[b])

    def fire_ids(c, b):
        ca, cb = ids_copies(c, b)
        ca.start()
        cb.start()

    def build_and_fire_gathers(c, b):
        ca, cb = ids_copies(c, b)
        ca.wait()
        cb.wait()
        g = base + c * K
        for i in range(K // L):
            pos = jnp.remainder(g + (L * i) + iota, SEQ)
            ci_v[b, pl.ds(L * i, L)] = 2 * pos + tt_v[b, pl.ds(L * i, L)]
        gather_copy(b).start()

    def tok_loop(b, apply_gb):
        @plsc.parallel_loop(0, K, unroll=4)
        def tok(t):
            ci = ci_v[b, pl.ds(t, L)][0]
            s = jnp.zeros((L,), jnp.float32)
            q = jnp.zeros((L,), jnp.float32)
            vs = []
            for j in range(HID // L):
                v = wrows[b, t, pl.ds(L * j, L)] + comb_v[ci, pl.ds(L * j, L)]
                vs.append(v)
                s = s + v
                q = q + v * v
            mean = jnp.sum(s) * (1.0 / HID)
            var = jnp.sum(q) * (1.0 / HID) - mean * mean
            r = _rsqrt_scalar(var + EPS)
            for j in range(HID // L):
                o = (vs[j] - mean) * r
                if apply_gb:
                    o = o * gregs[j] + bregs[j]
                obuf[b, t, pl.ds(L * j, L)] = o

    def compute(c, b):
        @pl.when(gb_trivial)
        def _():
            tok_loop(b, False)

        @pl.when(jnp.logical_not(gb_trivial))
        def _():
            tok_loop(b, True)

    fire_ids(0, 0)
    build_and_fire_gathers(0, 0)
    fire_ids(1, 1)

    def step(c, b, bn):
        gather_copy(b).wait()

        @pl.when(c + 1 < nchunks)
        def _():
            build_and_fire_gathers(c + 1, bn)

        @pl.when(c + 2 < nchunks)
        def _():
            fire_ids(c + 2, b)

        @pl.when(c >= 2)
        def _():
            out_copy(c - 2, b).wait()

        compute(c, b)
        out_copy(c, b).start()

    def pair(i, carry):
        step(2 * i, 0, 1)
        step(2 * i + 1, 1, 0)
        return carry

    lax.fori_loop(0, nchunks // 2, pair, 0)
    out_copy(nchunks - 2, 0).wait()
    out_copy(nchunks - 1, 1).wait()


def kernel(input_ids, token_type_ids, word_emb, pos_emb, tok_emb, gamma, beta):
    B, S = input_ids.shape
    V, H = word_emb.shape
    N = B * S
    ids = input_ids.reshape(N).astype(jnp.int32)
    tts = token_type_ids.reshape(N).astype(jnp.int32)
    comb = (pos_emb[:S, None, :] + tok_emb[None, :, :]).reshape(2 * S, H)

    mesh = plsc.VectorSubcoreMesh(core_axis_name="c", subcore_axis_name="s")
    run = pl.kernel(
        _body,
        out_type=jax.ShapeDtypeStruct((N, H), jnp.float32),
        mesh=mesh,
        compiler_params=pltpu.CompilerParams(needs_layout_passes=False),
        scratch_types=[
            pltpu.VMEM((2, K), jnp.int32),
            pltpu.VMEM((2, K + L), jnp.int32),
            pltpu.VMEM((2, K), jnp.int32),
            pltpu.VMEM((2, K, HID), jnp.float32),
            pltpu.VMEM((2, K, HID), jnp.float32),
            pltpu.VMEM((2 * SEQ, HID), jnp.float32),
            pltpu.VMEM((2, HID), jnp.float32),
            [pltpu.SemaphoreType.DMA, pltpu.SemaphoreType.DMA],
            [pltpu.SemaphoreType.DMA, pltpu.SemaphoreType.DMA],
            [pltpu.SemaphoreType.DMA, pltpu.SemaphoreType.DMA],
        ],
    )
    out = run(ids, tts, word_emb, comb, gamma, beta)
    return out.reshape(B, S, H)


# speedup vs baseline: 1.0365x; 1.0365x over previous
import jax
import jax.numpy as jnp
from jax import lax
from jax.experimental import pallas as pl
from jax.experimental.pallas import tpu as pltpu
from jax.experimental.pallas import tpu_sc as plsc

L = 16
NC = 2
NS = 16
NW = NC * NS
HID = 128
SEQ = 200
K = 128
EPS = 1e-12

def _rsqrt_scalar(x):
    i = lax.bitcast_convert_type(x, jnp.int32)
    i = jnp.int32(0x5F3759DF) - lax.shift_right_logical(i, 1)
    y = lax.bitcast_convert_type(i, jnp.float32)
    y = y * (1.5 - 0.5 * x * y * y)
    return y


def _body(ids_hbm, tt_hbm, word_hbm, comb_hbm, gamma_hbm, beta_hbm, out_hbm,
          idx_v, ci_v, tt_v, wrows, obuf, comb_v, gb_v,
          sem_i, sem_w, sem_o):
    wid = lax.axis_index("s") * NC + lax.axis_index("c")
    tokens_per_w = ids_hbm.shape[0] // NW
    nchunks = tokens_per_w // K
    base = wid * tokens_per_w

    pltpu.sync_copy(gamma_hbm, gb_v.at[0])
    pltpu.sync_copy(beta_hbm, gb_v.at[1])
    pltpu.sync_copy(comb_hbm, comb_v)
    gregs = [gb_v[0, pl.ds(L * j, L)] for j in range(HID // L)]
    bregs = [gb_v[1, pl.ds(L * j, L)] for j in range(HID // L)]
    iota = lax.iota(jnp.int32, L)

    acc = None
    for j in range(HID // L):
        ok = jnp.logical_and(gregs[j] == 1.0, bregs[j] == 0.0)
        acc = ok if acc is None else jnp.logical_and(acc, ok)
    gb_trivial = jnp.all(acc)

    def ids_copies(c, b):
        g = base + c * K
        return (pltpu.make_async_copy(ids_hbm.at[pl.ds(g, K)], idx_v.at[b],
                                      sem_i[b]),
                pltpu.make_async_copy(tt_hbm.at[pl.ds(g, K)], tt_v.at[b],
                                      sem_i[b]))

    def gather_copy(b):
        return pltpu.make_async_copy(word_hbm.at[idx_v.at[b]], wrows.at[b],
                                     sem_w[b])

    def out_copy(c, b):
        g = base + c * K
        return pltpu.make_async_copy(obuf.at[b], out_hbm.at[pl.ds(g, K)],
                                     sem_o[b])

    def fire_ids(c, b):
        ca, cb = ids_copies(c, b)
        ca.start()
        cb.start()

    def build_and_fire_gathers(c, b):
        ca, cb = ids_copies(c, b)
        ca.wait()
        cb.wait()
        g = base + c * K
        for i in range(K // L):
            pos = jnp.remainder(g + (L * i) + iota, SEQ)
            ci_v[b, pl.ds(L * i, L)] = 2 * pos + tt_v[b, pl.ds(L * i, L)]
        gather_copy(b).start()

    def tok_loop(b, apply_gb):
        @plsc.parallel_loop(0, K, unroll=4)
        def tok(t):
            ci = ci_v[b, pl.ds(t, L)][0]
            s = jnp.zeros((L,), jnp.float32)
            q = jnp.zeros((L,), jnp.float32)
            vs = []
            for j in range(HID // L):
                v = wrows[b, t, pl.ds(L * j, L)] + comb_v[ci, pl.ds(L * j, L)]
                vs.append(v)
                s = s + v
                q = q + v * v
            mean = jnp.sum(s) * (1.0 / HID)
            var = jnp.sum(q) * (1.0 / HID) - mean * mean
            r = _rsqrt_scalar(var + EPS)
            for j in range(HID // L):
                o = (vs[j] - mean) * r
                if apply_gb:
                    o = o * gregs[j] + bregs[j]
                obuf[b, t, pl.ds(L * j, L)] = o

    def compute(c, b):
        @pl.when(gb_trivial)
        def _():
            tok_loop(b, False)

        @pl.when(jnp.logical_not(gb_trivial))
        def _():
            tok_loop(b, True)

    fire_ids(0, 0)
    build_and_fire_gathers(0, 0)
    fire_ids(1, 1)

    def step(c, b, bn):
        gather_copy(b).wait()

        @pl.when(c + 1 < nchunks)
        def _():
            build_and_fire_gathers(c + 1, bn)

        @pl.when(c + 2 < nchunks)
        def _():
            fire_ids(c + 2, b)

        @pl.when(c >= 2)
        def _():
            out_copy(c - 2, b).wait()

        compute(c, b)
        out_copy(c, b).start()

    def pair(i, carry):
        step(2 * i, 0, 1)
        step(2 * i + 1, 1, 0)
        return carry

    lax.fori_loop(0, nchunks // 2, pair, 0)
    out_copy(nchunks - 2, 0).wait()
    out_copy(nchunks - 1, 1).wait()


def kernel(input_ids, token_type_ids, word_emb, pos_emb, tok_emb, gamma, beta):
    B, S = input_ids.shape
    V, H = word_emb.shape
    N = B * S
    ids = input_ids.reshape(N).astype(jnp.int32)
    tts = token_type_ids.reshape(N).astype(jnp.int32)
    comb = (pos_emb[:S, None, :] + tok_emb[None, :, :]).reshape(2 * S, H)

    mesh = plsc.VectorSubcoreMesh(core_axis_name="c", subcore_axis_name="s")
    run = pl.kernel(
        _body,
        out_type=jax.ShapeDtypeStruct((N, H), jnp.float32),
        mesh=mesh,
        compiler_params=pltpu.CompilerParams(needs_layout_passes=False),
        scratch_types=[
            pltpu.VMEM((2, K), jnp.int32),
            pltpu.VMEM((2, K + L), jnp.int32),
            pltpu.VMEM((2, K), jnp.int32),
            pltpu.VMEM((2, K, HID), jnp.float32),
            pltpu.VMEM((2, K, HID), jnp.float32),
            pltpu.VMEM((2 * SEQ, HID), jnp.float32),
            pltpu.VMEM((2, HID), jnp.float32),
            [pltpu.SemaphoreType.DMA, pltpu.SemaphoreType.DMA],
            [pltpu.SemaphoreType.DMA, pltpu.SemaphoreType.DMA],
            [pltpu.SemaphoreType.DMA, pltpu.SemaphoreType.DMA],
        ],
    )
    out = run(ids, tts, word_emb, comb, gamma, beta)
    return out.reshape(B, S, H)
